# baseline (speedup 1.0000x reference)
import functools

import jax
import jax.numpy as jnp
from jax import lax
from jax.experimental import pallas as pl
from jax.experimental.pallas import tpu as pltpu
from jax.experimental.pallas import tpu_sc as plsc

H = 128
NC, NS, LANES = 2, 16, 16
NW = NC * NS
CH = 128
DW = 16


def _mesh():
    return plsc.VectorSubcoreMesh(core_axis_name="c", subcore_axis_name="s")


@functools.lru_cache(None)
def _make_segsum(n_pad, kch):
    rpt = n_pad // NS

    def body(h, src, dst, zeros, out, acc, src_v, dst_v, rows, sem):
        c = lax.axis_index("c")
        s = lax.axis_index("s")
        wid = s * NC + c
        pltpu.sync_copy(zeros, acc.at[pl.ds(s * rpt, rpt)])
        pltpu.sync_copy(src.at[wid], src_v)
        pltpu.sync_copy(dst.at[wid], dst_v)
        plsc.subcore_barrier()

        def step(i, carry):
            pltpu.async_copy(h.at[src_v.at[i]], rows, sem).wait()
            pltpu.sync_copy(rows, acc.at[dst_v.at[i]], add=True)
            return carry

        lax.fori_loop(0, kch, step, 0)
        plsc.subcore_barrier()
        pltpu.sync_copy(acc.at[pl.ds(s * rpt, rpt)],
                        out.at[c, pl.ds(s * rpt, rpt)])

    return pl.kernel(
        body,
        out_type=jax.ShapeDtypeStruct((NC, n_pad, H), jnp.float32),
        mesh=_mesh(),
        scratch_types=[
            pltpu.VMEM_SHARED((n_pad, H), jnp.float32),
            pltpu.VMEM((kch, CH), jnp.int32),
            pltpu.VMEM((kch, CH), jnp.int32),
            pltpu.VMEM((CH, H), jnp.float32),
            pltpu.SemaphoreType.DMA,
        ],
    )


@functools.lru_cache(None)
def _make_pred(el_pad, kp):
    per_tile = kp * CH

    def body(ha, hb, ia, ib, out, iva, ivb, ra, rb, pacc, sem):
        c = lax.axis_index("c")
        s = lax.axis_index("s")
        wid = s * NC + c
        pltpu.sync_copy(ia.at[wid], iva)
        pltpu.sync_copy(ib.at[wid], ivb)

        def chunk(i, carry):
            pltpu.async_copy(ha.at[iva.at[i]], ra, sem).wait()
            pltpu.async_copy(hb.at[ivb.at[i]], rb, sem).wait()

            def row(r, carry2):
                accv = ra[r, pl.ds(0, LANES)] * rb[r, pl.ds(0, LANES)]
                for j in range(1, H // LANES):
                    accv = accv + (ra[r, pl.ds(j * LANES, LANES)] *
                                   rb[r, pl.ds(j * LANES, LANES)])
                pacc[r, :] = accv
                return carry2

            lax.fori_loop(0, CH, row, 0)
            pltpu.sync_copy(pacc, out.at[pl.ds(wid * per_tile + i * CH, CH)])
            return carry

        lax.fori_loop(0, kp, chunk, 0)

    return pl.kernel(
        body,
        out_type=jax.ShapeDtypeStruct((el_pad, LANES), jnp.float32),
        mesh=_mesh(),
        scratch_types=[
            pltpu.VMEM((kp, CH), jnp.int32),
            pltpu.VMEM((kp, CH), jnp.int32),
            pltpu.VMEM((CH, H), jnp.float32),
            pltpu.VMEM((CH, H), jnp.float32),
            pltpu.VMEM((CH, LANES), jnp.float32),
            pltpu.SemaphoreType.DMA,
        ],
    )


def _proj(x, w, b, emb, bm):
    n, d = x.shape

    def body(x_ref, w_ref, b_ref, e_ref, o_ref):
        y = lax.dot_general(x_ref[...], w_ref[...], (((1,), (1,)), ((), ())),
                            preferred_element_type=jnp.float32)
        o_ref[...] = y + b_ref[...] + e_ref[...]

    return pl.pallas_call(
        body,
        grid=(n // bm,),
        in_specs=[
            pl.BlockSpec((bm, d), lambda i: (i, 0)),
            pl.BlockSpec((H, d), lambda i: (0, 0)),
            pl.BlockSpec((1, H), lambda i: (0, 0)),
            pl.BlockSpec((bm, H), lambda i: (i, 0)),
        ],
        out_specs=pl.BlockSpec((bm, H), lambda i: (i, 0)),
        out_shape=jax.ShapeDtypeStruct((n, H), jnp.float32),
    )(x, w, b.reshape(1, H), emb)


def _combine(msgp, degp, hdst, wl_t, wr_t, b, bm):
    n = hdst.shape[0]

    def body(m_ref, dg_ref, h_ref, wl_ref, wr_ref, b_ref, y_ref, st_ref):
        m = m_ref[...]
        dg = dg_ref[...]
        deg = dg[0, :, 0:1] + dg[1, :, 0:1]
        agg = (m[0] + m[1]) * (1.0 / jnp.maximum(deg, 1.0))
        y = (jnp.dot(agg, wl_ref[...], preferred_element_type=jnp.float32)
             + jnp.dot(h_ref[...], wr_ref[...],
                       preferred_element_type=jnp.float32)
             + b_ref[...])
        y_ref[...] = y

        @pl.when(pl.program_id(0) == 0)
        def _():
            st_ref[...] = jnp.zeros_like(st_ref)

        s0 = jnp.sum(y, axis=0, keepdims=True)
        s1 = jnp.sum(y * y, axis=0, keepdims=True)
        st_ref[...] += jnp.concatenate([s0, s1], axis=0)

    return pl.pallas_call(
        body,
        grid=(n // bm,),
        in_specs=[
            pl.BlockSpec((NC, bm, H), lambda i: (0, i, 0)),
            pl.BlockSpec((NC, bm, H), lambda i: (0, i, 0)),
            pl.BlockSpec((bm, H), lambda i: (i, 0)),
            pl.BlockSpec((H, H), lambda i: (0, 0)),
            pl.BlockSpec((H, H), lambda i: (0, 0)),
            pl.BlockSpec((1, H), lambda i: (0, 0)),
        ],
        out_specs=(pl.BlockSpec((bm, H), lambda i: (i, 0)),
                   pl.BlockSpec((2, H), lambda i: (0, 0))),
        out_shape=(jax.ShapeDtypeStruct((n, H), jnp.float32),
                   jax.ShapeDtypeStruct((2, H), jnp.float32)),
    )(msgp, degp, hdst, wl_t, wr_t, b.reshape(1, H))


def _bn_relu(y, stats, g, b, bm):
    n = y.shape[0]
    inv_n = 1.0 / n

    def body(y_ref, st_ref, g_ref, b_ref, o_ref):
        st = st_ref[...]
        m = st[0:1] * inv_n
        v = st[1:2] * inv_n - m * m
        o_ref[...] = jnp.maximum(
            g_ref[...] * (y_ref[...] - m) * lax.rsqrt(v + 1e-5) + b_ref[...],
            0.0)

    return pl.pallas_call(
        body,
        grid=(n // bm,),
        in_specs=[
            pl.BlockSpec((bm, H), lambda i: (i, 0)),
            pl.BlockSpec((2, H), lambda i: (0, 0)),
            pl.BlockSpec((1, H), lambda i: (0, 0)),
            pl.BlockSpec((1, H), lambda i: (0, 0)),
        ],
        out_specs=pl.BlockSpec((bm, H), lambda i: (i, 0)),
        out_shape=jax.ShapeDtypeStruct((n, H), jnp.float32),
    )(y, stats, g.reshape(1, H), b.reshape(1, H))


def _rowsum16(part, n_out, bm):
    def body(p_ref, o_ref):
        o_ref[...] = jnp.sum(p_ref[...], axis=1, keepdims=True)

    return pl.pallas_call(
        body,
        grid=(n_out // bm,),
        in_specs=[pl.BlockSpec((bm, LANES), lambda i: (i, 0))],
        out_specs=pl.BlockSpec((bm, 1), lambda i: (i, 0)),
        out_shape=jax.ShapeDtypeStruct((n_out, 1), jnp.float32),
    )(part)


def _ceil_to(x, m):
    return -(-x // m) * m


def kernel(x_st, x_vc, node_ids_st, node_ids_vc, edge_index, edge_label_index,
           params):
    p = params
    n_st, d_st = x_st.shape
    n_vc, d_vc = x_vc.shape
    e = edge_index.shape[1]
    el = edge_label_index.shape[1]
    n = max(n_st, n_vc)
    n_pad = _ceil_to(n + 1, 256)
    dummy = n

    kch = -(-e // (NW * CH))
    e_pad = NW * kch * CH

    def pad_idx(a, fill):
        return jnp.concatenate(
            [a, jnp.full((e_pad - e,), fill, jnp.int32)]).reshape(NW, kch, CH)

    s_i, d_i = edge_index[0], edge_index[1]
    s_g = pad_idx(s_i, 0)
    d_g = pad_idx(d_i, 0)
    s_s = pad_idx(s_i, dummy)
    d_s = pad_idx(d_i, dummy)

    zeros_h = jnp.zeros((n_pad // NS, H), jnp.float32)

    segsum = _make_segsum(n_pad, kch)

    h_st = _proj(x_st, p['st_lin_W'], p['st_lin_b'],
                 p['emb_st'], 400)
    h_vc = _proj(x_vc, p['vc_lin_W'], p['vc_lin_b'],
                 p['emb_vc'], 400)
    ones_tbl = jnp.ones((8, H), jnp.float32)
    z_idx = jnp.zeros((NW, kch, CH), jnp.int32)
    degp_s = segsum(ones_tbl, z_idx, s_s, zeros_h)
    degp_d = segsum(ones_tbl, z_idx, d_s, zeros_h)

    for l, use_bn in ((1, True), (2, True), (3, False)):
        msgp_v = segsum(h_st, s_g, d_s, zeros_h)
        msgp_s = segsum(h_vc, d_g, s_s, zeros_h)
        nv, st_v = _combine(msgp_v, degp_d, h_vc,
                            p['c%d_st2vc_Wl' % l].T, p['c%d_st2vc_Wr' % l].T,
                            p['c%d_st2vc_b' % l], 2000)
        ns, st_s = _combine(msgp_s, degp_s, h_st,
                            p['c%d_vc2st_Wl' % l].T, p['c%d_vc2st_Wr' % l].T,
                            p['c%d_vc2st_b' % l], 2000)
        if use_bn:
            nv = _bn_relu(nv, st_v, p['bn%d_vc_g' % l], p['bn%d_vc_b' % l],
                          2000)
            ns = _bn_relu(ns, st_s, p['bn%d_st_g' % l], p['bn%d_st_b' % l],
                          2000)
        h_vc, h_st = nv, ns

    kp = -(-el // (NW * CH))
    el_pad = NW * kp * CH

    def pad_el(a):
        return jnp.concatenate(
            [a, jnp.zeros((el_pad - el,), jnp.int32)]).reshape(NW, kp, CH)

    pred_k = _make_pred(el_pad, kp)
    part = pred_k(h_st, h_vc, pad_el(edge_label_index[0]),
                  pad_el(edge_label_index[1]))
    pred = _rowsum16(part, el, 4000).reshape(el)

    return pred, h_st, h_vc


# speedup vs baseline: 7.6009x; 7.6009x over previous
import functools

import jax
import jax.numpy as jnp
from jax import lax
from jax.experimental import pallas as pl
from jax.experimental.pallas import tpu as pltpu
from jax.experimental.pallas import tpu_sc as plsc

H = 128
NC, NS, LANES = 2, 16, 16
NW = NC * NS
CH = 128
DW = 16


def _mesh():
    return plsc.VectorSubcoreMesh(core_axis_name="c", subcore_axis_name="s")


@functools.lru_cache(None)
def _make_segsum(n_pad, kch):
    rpt = n_pad // NS

    def body(h, src, dst, zeros, out, acc, src_v, dst_v, rows, sem):
        c = lax.axis_index("c")
        s = lax.axis_index("s")
        wid = s * NC + c
        pltpu.sync_copy(zeros, acc.at[pl.ds(s * rpt, rpt)])
        pltpu.sync_copy(src.at[wid], src_v)
        pltpu.sync_copy(dst.at[wid], dst_v)
        plsc.subcore_barrier()

        def step(i, carry):
            pltpu.async_copy(h.at[src_v.at[i]], rows, sem).wait()
            pltpu.sync_copy(rows, acc.at[dst_v.at[i]], add=True)
            return carry

        lax.fori_loop(0, kch, step, 0)
        plsc.subcore_barrier()
        pltpu.sync_copy(acc.at[pl.ds(s * rpt, rpt)],
                        out.at[c, pl.ds(s * rpt, rpt)])

    return pl.kernel(
        body,
        out_type=jax.ShapeDtypeStruct((NC, n_pad, H), jnp.float32),
        mesh=_mesh(),
        scratch_types=[
            pltpu.VMEM_SHARED((n_pad, H), jnp.float32),
            pltpu.VMEM((kch, CH), jnp.int32),
            pltpu.VMEM((kch, CH), jnp.int32),
            pltpu.VMEM((CH, H), jnp.float32),
            pltpu.SemaphoreType.DMA,
        ],
    )


@functools.lru_cache(None)
def _make_pred(el_pad, kp):
    per_tile = kp * CH

    def body(ha, hb, ia, ib, out, iva, ivb, ra, rb, pacc, sem):
        c = lax.axis_index("c")
        s = lax.axis_index("s")
        wid = s * NC + c
        pltpu.sync_copy(ia.at[wid], iva)
        pltpu.sync_copy(ib.at[wid], ivb)

        def chunk(i, carry):
            pltpu.async_copy(ha.at[iva.at[i]], ra, sem).wait()
            pltpu.async_copy(hb.at[ivb.at[i]], rb, sem).wait()

            def row(r, carry2):
                accv = ra[r, pl.ds(0, LANES)] * rb[r, pl.ds(0, LANES)]
                for j in range(1, H // LANES):
                    accv = accv + (ra[r, pl.ds(j * LANES, LANES)] *
                                   rb[r, pl.ds(j * LANES, LANES)])
                pacc[r, :] = accv
                return carry2

            lax.fori_loop(0, CH, row, 0)
            pltpu.sync_copy(pacc, out.at[pl.ds(wid * per_tile + i * CH, CH)])
            return carry

        lax.fori_loop(0, kp, chunk, 0)

    return pl.kernel(
        body,
        out_type=jax.ShapeDtypeStruct((el_pad, LANES), jnp.float32),
        mesh=_mesh(),
        scratch_types=[
            pltpu.VMEM((kp, CH), jnp.int32),
            pltpu.VMEM((kp, CH), jnp.int32),
            pltpu.VMEM((CH, H), jnp.float32),
            pltpu.VMEM((CH, H), jnp.float32),
            pltpu.VMEM((CH, LANES), jnp.float32),
            pltpu.SemaphoreType.DMA,
        ],
    )


def _proj(x, w, b, emb, bm):
    n, d = x.shape

    def body(x_ref, w_ref, b_ref, e_ref, o_ref):
        y = lax.dot_general(x_ref[...], w_ref[...], (((1,), (1,)), ((), ())),
                            preferred_element_type=jnp.float32)
        o_ref[...] = y + b_ref[...] + e_ref[...]

    return pl.pallas_call(
        body,
        grid=(n // bm,),
        in_specs=[
            pl.BlockSpec((bm, d), lambda i: (i, 0)),
            pl.BlockSpec((H, d), lambda i: (0, 0)),
            pl.BlockSpec((1, H), lambda i: (0, 0)),
            pl.BlockSpec((bm, H), lambda i: (i, 0)),
        ],
        out_specs=pl.BlockSpec((bm, H), lambda i: (i, 0)),
        out_shape=jax.ShapeDtypeStruct((n, H), jnp.float32),
    )(x, w, b.reshape(1, H), emb)


def _combine(msgp, degp, hdst, wl_t, wr_t, b, bm):
    n = hdst.shape[0]

    def body(m_ref, dg_ref, h_ref, wl_ref, wr_ref, b_ref, y_ref, st_ref):
        m = m_ref[...]
        dg = dg_ref[...]
        deg = dg[0, :, 0:1] + dg[1, :, 0:1]
        agg = (m[0] + m[1]) * (1.0 / jnp.maximum(deg, 1.0))
        y = (jnp.dot(agg, wl_ref[...], preferred_element_type=jnp.float32)
             + jnp.dot(h_ref[...], wr_ref[...],
                       preferred_element_type=jnp.float32)
             + b_ref[...])
        y_ref[...] = y

        @pl.when(pl.program_id(0) == 0)
        def _():
            st_ref[...] = jnp.zeros_like(st_ref)

        s0 = jnp.sum(y, axis=0, keepdims=True)
        s1 = jnp.sum(y * y, axis=0, keepdims=True)
        st_ref[...] += jnp.concatenate([s0, s1], axis=0)

    return pl.pallas_call(
        body,
        grid=(n // bm,),
        in_specs=[
            pl.BlockSpec((NC, bm, H), lambda i: (0, i, 0)),
            pl.BlockSpec((NC, bm, H), lambda i: (0, i, 0)),
            pl.BlockSpec((bm, H), lambda i: (i, 0)),
            pl.BlockSpec((H, H), lambda i: (0, 0)),
            pl.BlockSpec((H, H), lambda i: (0, 0)),
            pl.BlockSpec((1, H), lambda i: (0, 0)),
        ],
        out_specs=(pl.BlockSpec((bm, H), lambda i: (i, 0)),
                   pl.BlockSpec((2, H), lambda i: (0, 0))),
        out_shape=(jax.ShapeDtypeStruct((n, H), jnp.float32),
                   jax.ShapeDtypeStruct((2, H), jnp.float32)),
    )(msgp, degp, hdst, wl_t, wr_t, b.reshape(1, H))


def _bn_relu(y, stats, g, b, bm):
    n = y.shape[0]
    inv_n = 1.0 / n

    def body(y_ref, st_ref, g_ref, b_ref, o_ref):
        st = st_ref[...]
        m = st[0:1] * inv_n
        v = st[1:2] * inv_n - m * m
        o_ref[...] = jnp.maximum(
            g_ref[...] * (y_ref[...] - m) * lax.rsqrt(v + 1e-5) + b_ref[...],
            0.0)

    return pl.pallas_call(
        body,
        grid=(n // bm,),
        in_specs=[
            pl.BlockSpec((bm, H), lambda i: (i, 0)),
            pl.BlockSpec((2, H), lambda i: (0, 0)),
            pl.BlockSpec((1, H), lambda i: (0, 0)),
            pl.BlockSpec((1, H), lambda i: (0, 0)),
        ],
        out_specs=pl.BlockSpec((bm, H), lambda i: (i, 0)),
        out_shape=jax.ShapeDtypeStruct((n, H), jnp.float32),
    )(y, stats, g.reshape(1, H), b.reshape(1, H))


def _rowsum16(part, n_out, bm):
    def body(p_ref, o_ref):
        o_ref[...] = jnp.sum(p_ref[...], axis=1, keepdims=True)

    return pl.pallas_call(
        body,
        grid=(n_out // bm,),
        in_specs=[pl.BlockSpec((bm, LANES), lambda i: (i, 0))],
        out_specs=pl.BlockSpec((bm, 1), lambda i: (i, 0)),
        out_shape=jax.ShapeDtypeStruct((n_out, 1), jnp.float32),
    )(part)


def _ceil_to(x, m):
    return -(-x // m) * m


def kernel(x_st, x_vc, node_ids_st, node_ids_vc, edge_index, edge_label_index,
           params):
    p = params
    n_st, d_st = x_st.shape
    n_vc, d_vc = x_vc.shape
    e = edge_index.shape[1]
    el = edge_label_index.shape[1]
    n = max(n_st, n_vc)
    n_pad = _ceil_to(n + 1, 256)
    dummy = n

    kch = -(-e // (NW * CH))
    e_pad = NW * kch * CH

    def pad_idx(a, fill):
        return jnp.concatenate(
            [a, jnp.full((e_pad - e,), fill, jnp.int32)]).reshape(NW, kch, CH)

    s_i, d_i = edge_index[0], edge_index[1]
    s_g = pad_idx(s_i, 0)
    d_g = pad_idx(d_i, 0)
    s_s = pad_idx(s_i, dummy)
    d_s = pad_idx(d_i, dummy)

    zeros_h = jnp.zeros((n_pad // NS, H), jnp.float32)

    segsum = _make_segsum(n_pad, kch)

    h_st = _proj(x_st, p['st_lin_W'], p['st_lin_b'],
                 p['emb_st'], 400)
    h_vc = _proj(x_vc, p['vc_lin_W'], p['vc_lin_b'],
                 p['emb_vc'], 400)
    ones_tbl = jnp.ones((n, H), jnp.float32)
    degp_s = segsum(ones_tbl, d_g, s_s, zeros_h)
    degp_d = segsum(ones_tbl, s_g, d_s, zeros_h)

    for l, use_bn in ((1, True), (2, True), (3, False)):
        msgp_v = segsum(h_st, s_g, d_s, zeros_h)
        msgp_s = segsum(h_vc, d_g, s_s, zeros_h)
        nv, st_v = _combine(msgp_v, degp_d, h_vc,
                            p['c%d_st2vc_Wl' % l].T, p['c%d_st2vc_Wr' % l].T,
                            p['c%d_st2vc_b' % l], 2000)
        ns, st_s = _combine(msgp_s, degp_s, h_st,
                            p['c%d_vc2st_Wl' % l].T, p['c%d_vc2st_Wr' % l].T,
                            p['c%d_vc2st_b' % l], 2000)
        if use_bn:
            nv = _bn_relu(nv, st_v, p['bn%d_vc_g' % l], p['bn%d_vc_b' % l],
                          2000)
            ns = _bn_relu(ns, st_s, p['bn%d_st_g' % l], p['bn%d_st_b' % l],
                          2000)
        h_vc, h_st = nv, ns

    kp = -(-el // (NW * CH))
    el_pad = NW * kp * CH

    def pad_el(a):
        return jnp.concatenate(
            [a, jnp.zeros((el_pad - el,), jnp.int32)]).reshape(NW, kp, CH)

    pred_k = _make_pred(el_pad, kp)
    part = pred_k(h_st, h_vc, pad_el(edge_label_index[0]),
                  pad_el(edge_label_index[1]))
    pred = _rowsum16(part, el, 4000).reshape(el)

    return pred, h_st, h_vc
